# baseline (speedup 1.0000x reference)
import functools

import jax
import jax.numpy as jnp
from jax import lax
from jax.experimental import pallas as pl
from jax.experimental.pallas import tpu as pltpu
from jax.experimental.pallas import tpu_sc as plsc

DIM = 256
CODES = 8192
TOKENS = 16384

M_BLOCK = 1024
N_BLOCK = 1024
M_TILES = TOKENS // M_BLOCK
N_TILES = CODES // N_BLOCK

SC_CORES = 2
SC_SUBCORES = 16
SC_WORKERS = SC_CORES * SC_SUBCORES
ROWS_PER_WORKER = TOKENS // SC_WORKERS
GATHER_CHUNK = 128

_BIG = 2**30


_CHUNKS = ((0, 2736), (2736, 2736), (5472, 2720))


def _argmin_dist_body(x_ref, e_ref, x2_ref, e2_ref, out_ref):
    xd = x_ref[...] + x_ref[...]
    x2 = x2_ref[0]

    bv = None
    bi = None
    for off, size in _CHUNKS:
        e = e_ref[pl.ds(off, size), :]
        e2 = e2_ref[pl.ds(off, size), :]
        xe2 = lax.dot_general(e, xd, (((1,), (1,)), ((), ())),
                              preferred_element_type=jnp.float32)
        dist = (xe2 - x2) - e2

        m = jnp.max(dist, axis=0, keepdims=True)
        idx = (jnp.argmax(dist, axis=0).astype(jnp.int32)[None, :] + off)

        if bv is None:
            bv, bi = m, idx
        else:
            better = m > bv
            bi = jnp.where(better, idx, bi)
            bv = jnp.maximum(bv, m)
        bv = bv.astype(jnp.bfloat16).astype(jnp.float32)

    out_ref[0] = bi


def _nearest_code_indices(flat_x, embed, x2, e2):
    tokens = flat_x.shape[0]
    m_tiles = tokens // M_BLOCK
    x2_3d = x2.reshape(m_tiles, 1, M_BLOCK)
    e2_2d = e2.reshape(CODES, 1)
    out = pl.pallas_call(
        _argmin_dist_body,
        grid=(m_tiles,),
        in_specs=[
            pl.BlockSpec((M_BLOCK, DIM), lambda i: (i, 0)),
            pl.BlockSpec((CODES, DIM), lambda i: (0, 0)),
            pl.BlockSpec((1, 1, M_BLOCK), lambda i: (i, 0, 0)),
            pl.BlockSpec((CODES, 1), lambda i: (0, 0)),
        ],
        out_specs=pl.BlockSpec((1, 1, M_BLOCK), lambda i: (i, 0, 0)),
        out_shape=jax.ShapeDtypeStruct((m_tiles, 1, M_BLOCK), jnp.int32),
    )(flat_x, embed, x2_3d, e2_2d)
    return out.reshape(tokens)


@functools.cache
def _make_sc_gather(tokens):
    mesh = plsc.VectorSubcoreMesh(core_axis_name="c", subcore_axis_name="s")
    rows_per_worker = tokens // SC_WORKERS
    chunk = min(rows_per_worker, GATHER_CHUNK)
    n_chunks = rows_per_worker // chunk

    @functools.partial(
        pl.kernel,
        mesh=mesh,
        out_type=jax.ShapeDtypeStruct((tokens, DIM), jnp.float32),
        scratch_types=[
            pltpu.VMEM((n_chunks, chunk), jnp.int32),
            pltpu.VMEM((2, chunk, DIM), jnp.float32),
            pltpu.SemaphoreType.DMA,
            pltpu.SemaphoreType.DMA,
            pltpu.SemaphoreType.DMA,
        ],
    )
    def gather_rows(table_hbm, idx_hbm, out_hbm, idx_v, rows_v, gsem, s0, s1):
        wid = lax.axis_index("s") * SC_CORES + lax.axis_index("c")
        base = wid * rows_per_worker
        pltpu.sync_copy(idx_hbm.at[pl.ds(wid * n_chunks, n_chunks)], idx_v)
        ssem = (s0, s1)
        stores = [None, None]
        for c in range(n_chunks):
            b = c & 1
            if stores[b] is not None:
                stores[b].wait()
            pltpu.async_copy(table_hbm.at[idx_v.at[c]], rows_v.at[b],
                             gsem).wait()
            stores[b] = pltpu.async_copy(
                rows_v.at[b], out_hbm.at[pl.ds(base + c * chunk, chunk)],
                ssem[b])
        for st in stores:
            if st is not None:
                st.wait()

    return gather_rows


def kernel(x, embed):
    shape = x.shape
    flat_x = x.reshape(-1, shape[-1])
    x2 = jnp.sum(flat_x ** 2, axis=1)
    e2 = jnp.sum(embed ** 2, axis=1)
    idx = _nearest_code_indices(flat_x, embed, x2, e2)
    idx2d = idx.reshape(-1, GATHER_CHUNK)
    quantize = _make_sc_gather(TOKENS)(embed, idx2d).reshape(shape)
    num_replace = jnp.array(0, dtype=jnp.int32)
    return (quantize, num_replace)


# speedup vs baseline: 1.0087x; 1.0087x over previous
import functools

import jax
import jax.numpy as jnp
from jax import lax
from jax.experimental import pallas as pl
from jax.experimental.pallas import tpu as pltpu
from jax.experimental.pallas import tpu_sc as plsc

DIM = 256
CODES = 8192
TOKENS = 16384

M_BLOCK = 1024
N_BLOCK = 1024
M_TILES = TOKENS // M_BLOCK
N_TILES = CODES // N_BLOCK

SC_CORES = 2
SC_SUBCORES = 16
SC_WORKERS = SC_CORES * SC_SUBCORES
ROWS_PER_WORKER = TOKENS // SC_WORKERS
GATHER_CHUNK = 256

_BIG = 2**30


_CHUNKS = ((0, 2736), (2736, 2736), (5472, 2720))


def _argmin_dist_body(x_ref, e_ref, x2_ref, e2_ref, out_ref):
    xd = x_ref[...] + x_ref[...]
    x2 = x2_ref[0]

    bv = None
    bi = None
    for off, size in _CHUNKS:
        e = e_ref[pl.ds(off, size), :]
        e2 = e2_ref[pl.ds(off, size), :]
        xe2 = lax.dot_general(e, xd, (((1,), (1,)), ((), ())),
                              preferred_element_type=jnp.float32)
        dist = (xe2 - x2) - e2

        m = jnp.max(dist, axis=0, keepdims=True)
        idx = (jnp.argmax(dist, axis=0).astype(jnp.int32)[None, :] + off)

        if bv is None:
            bv, bi = m, idx
        else:
            better = m > bv
            bi = jnp.where(better, idx, bi)
            bv = jnp.maximum(bv, m)
        bv = bv.astype(jnp.bfloat16).astype(jnp.float32)

    out_ref[0] = bi


def _nearest_code_indices(flat_x, embed, x2, e2):
    tokens = flat_x.shape[0]
    m_tiles = tokens // M_BLOCK
    x2_3d = x2.reshape(m_tiles, 1, M_BLOCK)
    e2_2d = e2.reshape(CODES, 1)
    out = pl.pallas_call(
        _argmin_dist_body,
        grid=(m_tiles,),
        in_specs=[
            pl.BlockSpec((M_BLOCK, DIM), lambda i: (i, 0)),
            pl.BlockSpec((CODES, DIM), lambda i: (0, 0)),
            pl.BlockSpec((1, 1, M_BLOCK), lambda i: (i, 0, 0)),
            pl.BlockSpec((CODES, 1), lambda i: (0, 0)),
        ],
        out_specs=pl.BlockSpec((1, 1, M_BLOCK), lambda i: (i, 0, 0)),
        out_shape=jax.ShapeDtypeStruct((m_tiles, 1, M_BLOCK), jnp.int32),
    )(flat_x, embed, x2_3d, e2_2d)
    return out.reshape(tokens)


@functools.cache
def _make_sc_gather(tokens):
    mesh = plsc.VectorSubcoreMesh(core_axis_name="c", subcore_axis_name="s")
    rows_per_worker = tokens // SC_WORKERS
    chunk = min(rows_per_worker, GATHER_CHUNK)
    n_chunks = rows_per_worker // chunk

    @functools.partial(
        pl.kernel,
        mesh=mesh,
        out_type=jax.ShapeDtypeStruct((tokens, DIM), jnp.float32),
        scratch_types=[
            pltpu.VMEM((chunk,), jnp.int32),
            pltpu.VMEM((chunk, DIM), jnp.float32),
            pltpu.SemaphoreType.DMA,
        ],
    )
    def gather_rows(table_hbm, idx_hbm, out_hbm, idx_v, rows_v, sem):
        wid = lax.axis_index("s") * SC_CORES + lax.axis_index("c")
        for c in range(n_chunks):
            base = wid * rows_per_worker + c * chunk
            pltpu.sync_copy(idx_hbm.at[pl.ds(base, chunk)], idx_v)
            pltpu.async_copy(table_hbm.at[idx_v], rows_v, sem).wait()
            pltpu.sync_copy(rows_v, out_hbm.at[pl.ds(base, chunk)])

    return gather_rows


def kernel(x, embed):
    shape = x.shape
    flat_x = x.reshape(-1, shape[-1])
    x2 = jnp.sum(flat_x ** 2, axis=1)
    e2 = jnp.sum(embed ** 2, axis=1)
    idx = _nearest_code_indices(flat_x, embed, x2, e2)
    quantize = _make_sc_gather(TOKENS)(embed, idx).reshape(shape)
    num_replace = jnp.array(0, dtype=jnp.int32)
    return (quantize, num_replace)
